# Initial kernel scaffold; baseline (speedup 1.0000x reference)
import jax
import jax.numpy as jnp
from jax.experimental import pallas as pl


def kernel(x, edge_index, y, batch, W, b):
    raise NotImplementedError("write your pallas kernel here")


# speedup vs baseline: 14.7459x; 14.7459x over previous
import functools

import jax
import jax.numpy as jnp
from jax import lax
from jax.experimental import pallas as pl
from jax.experimental.pallas import tpu as pltpu
from jax.experimental.pallas import tpu_sc as plsc

NC = 2
NS = 16
LANES = 16
NW = NC * NS
K = 128


def _sc_mesh():
    return plsc.VectorSubcoreMesh(
        core_axis_name="c", subcore_axis_name="s", num_cores=NC, num_subcores=NS
    )


def _make_deg_fn(E_pad, N_pad):
    PT = E_pad // NW
    n_chunks = PT // K

    def body(col_hbm, out_hbm, hist, cidx):
        c = lax.axis_index("c")
        s = lax.axis_index("s")
        tid = c * NS + s

        def zfill(i, carry):
            hist[pl.ds(i * LANES, LANES)] = jnp.zeros((LANES,), jnp.float32)
            return carry

        lax.fori_loop(0, N_pad // LANES, zfill, 0)

        ebase = tid * PT
        ones_v = jnp.ones((LANES,), jnp.float32)

        def chunk(j, carry):
            pltpu.sync_copy(col_hbm.at[pl.ds(ebase + j * K, K)], cidx)
            for t in range(K // LANES):
                iv = cidx[pl.ds(t * LANES, LANES)]
                plsc.addupdate_scatter(hist, [iv], ones_v)
            return carry

        lax.fori_loop(0, n_chunks, chunk, 0)
        pltpu.sync_copy(hist, out_hbm.at[pl.ds(tid * N_pad, N_pad)])

    return pl.kernel(
        body,
        out_type=jax.ShapeDtypeStruct((NW * N_pad,), jnp.float32),
        mesh=_sc_mesh(),
        compiler_params=pltpu.CompilerParams(needs_layout_passes=False),
        scratch_types=[
            pltpu.VMEM((N_pad,), jnp.float32),
            pltpu.VMEM((K,), jnp.int32),
        ],
    )


def _make_scatter_fn(E_pad, N_pad, C):
    PT = E_pad // NW
    n_chunks = PT // K
    R = N_pad // NS

    def body(g_hbm, row_hbm, col_hbm, out_hbm, rows_v, ridx, cidx, acc, sem):
        c = lax.axis_index("c")
        s = lax.axis_index("s")
        tid = c * NS + s

        def zfill(i, carry):
            for t in range(C // LANES):
                rows_v[i, pl.ds(t * LANES, LANES)] = jnp.zeros((LANES,), jnp.float32)
            return carry

        lax.fori_loop(0, K, zfill, 0)
        base_row = s * R
        for k in range(R // K):
            pltpu.sync_copy(rows_v, acc.at[pl.ds(base_row + k * K, K)])
        plsc.subcore_barrier()

        ebase = tid * PT

        def chunk(j, carry):
            pltpu.sync_copy(row_hbm.at[pl.ds(ebase + j * K, K)], ridx)
            pltpu.sync_copy(col_hbm.at[pl.ds(ebase + j * K, K)], cidx)
            pltpu.async_copy(g_hbm.at[ridx], rows_v, sem).wait()
            pltpu.sync_copy(rows_v, acc.at[cidx], add=True)
            return carry

        lax.fori_loop(0, n_chunks, chunk, 0)
        plsc.subcore_barrier()
        pltpu.sync_copy(
            acc.at[pl.ds(base_row, R)],
            out_hbm.at[pl.ds(c * N_pad + base_row, R)],
        )

    return pl.kernel(
        body,
        out_type=jax.ShapeDtypeStruct((NC * N_pad, C), jnp.float32),
        mesh=_sc_mesh(),
        scratch_types=[
            pltpu.VMEM((K, C), jnp.float32),
            pltpu.VMEM((K,), jnp.int32),
            pltpu.VMEM((K,), jnp.int32),
            pltpu.VMEM_SHARED((N_pad, C), jnp.float32),
            pltpu.SemaphoreType.DMA,
        ],
    )


def _dinv_from_hists(hblk):
    deg = jnp.sum(hblk, axis=0)
    return jnp.where(deg > 0, lax.rsqrt(jnp.maximum(deg, 1.0)), 0.0)


def _linear_scale(x, W, hists, BLK=1024):
    N, C = x.shape

    def body(x_ref, w_ref, h_ref, g_ref):
        dinv = _dinv_from_hists(h_ref[...])
        h = lax.dot_general(
            x_ref[...], w_ref[...], (((1,), (1,)), ((), ())),
            preferred_element_type=jnp.float32,
        )
        g_ref[...] = h * dinv[:, None]

    return pl.pallas_call(
        body,
        grid=(N // BLK,),
        in_specs=[
            pl.BlockSpec((BLK, C), lambda i: (i, 0)),
            pl.BlockSpec((C, C), lambda i: (0, 0)),
            pl.BlockSpec((NW, BLK), lambda i: (0, i)),
        ],
        out_specs=pl.BlockSpec((BLK, C), lambda i: (i, 0)),
        out_shape=jax.ShapeDtypeStruct((N, C), jnp.float32),
    )(x, W, hists)


def _pool_loss(parts, hists, batch3, y2, b2, B, BLK=1024):
    _, _, C = parts.shape
    nblk = batch3.shape[0]

    def body(p0, p1, h_ref, bt, y_ref, b_ref, pooled_ref, loss_ref, pacc, cacc):
        i = pl.program_id(0)

        @pl.when(i == 0)
        def _():
            pacc[...] = jnp.zeros_like(pacc)
            cacc[...] = jnp.zeros_like(cacc)

        dinv = _dinv_from_hists(h_ref[...])
        outb = jnp.maximum(
            (p0[0] + p1[0]) * dinv[:, None] + b_ref[...], 0.0
        )
        bt_v = bt[0, 0, :]
        onehot = (
            bt_v[None, :] == lax.broadcasted_iota(jnp.int32, (B, BLK), 0)
        ).astype(jnp.float32)
        pacc[...] += lax.dot_general(
            onehot, outb, (((1,), (0,)), ((), ())),
            preferred_element_type=jnp.float32,
        )
        cacc[...] += jnp.sum(onehot, axis=1, keepdims=True)

        @pl.when(i == nblk - 1)
        def _():
            pooled = pacc[...] / jnp.maximum(cacc[...], 1.0)
            pooled_ref[...] = pooled
            m = jnp.max(pooled, axis=1, keepdims=True)
            lse = m + jnp.log(jnp.sum(jnp.exp(pooled - m), axis=1, keepdims=True))
            logp = pooled - lse
            oy = (
                lax.broadcasted_iota(jnp.int32, (B, C), 1) == y_ref[0][:, None]
            ).astype(jnp.float32)
            nll = -jnp.sum(logp * oy, axis=1, keepdims=True)
            loss_ref[...] = jnp.mean(nll).reshape(1, 1)

    return pl.pallas_call(
        body,
        grid=(nblk,),
        in_specs=[
            pl.BlockSpec((1, BLK, C), lambda i: (0, i, 0)),
            pl.BlockSpec((1, BLK, C), lambda i: (1, i, 0)),
            pl.BlockSpec((NW, BLK), lambda i: (0, i)),
            pl.BlockSpec((1, 1, BLK), lambda i: (i, 0, 0)),
            pl.BlockSpec((1, B), lambda i: (0, 0)),
            pl.BlockSpec((1, C), lambda i: (0, 0)),
        ],
        out_specs=[
            pl.BlockSpec((B, C), lambda i: (0, 0)),
            pl.BlockSpec((1, 1), lambda i: (0, 0)),
        ],
        out_shape=[
            jax.ShapeDtypeStruct((B, C), jnp.float32),
            jax.ShapeDtypeStruct((1, 1), jnp.float32),
        ],
        scratch_shapes=[
            pltpu.VMEM((B, C), jnp.float32),
            pltpu.VMEM((B, 1), jnp.float32),
        ],
    )(parts, parts, hists, batch3, y2, b2)


def kernel(x, edge_index, y, batch, W, b):
    N, C = x.shape
    E = edge_index.shape[1]
    B = y.shape[0]

    row = edge_index[0].astype(jnp.int32)
    col = edge_index[1].astype(jnp.int32)

    PT = -(-E // (NW * K)) * K
    E_pad = PT * NW
    N_pad = -(-(N + 1) // (NS * K)) * (NS * K)

    pad = E_pad - E
    row_p = jnp.concatenate([row, jnp.zeros((pad,), jnp.int32)])
    col_p = jnp.concatenate([col, jnp.full((pad,), N, jnp.int32)])

    hists = _make_deg_fn(E_pad, N_pad)(col_p).reshape(NW, N_pad)
    x_p = jnp.concatenate([x, jnp.zeros((N_pad - N, C), x.dtype)])
    g = _linear_scale(x_p, W, hists)
    parts = _make_scatter_fn(E_pad, N_pad, C)(g, row_p, col_p).reshape(NC, N_pad, C)

    BLK = 1024
    batch_p = jnp.concatenate(
        [batch.astype(jnp.int32), jnp.full((N_pad - N,), B, jnp.int32)]
    )
    batch3 = batch_p.reshape(N_pad // BLK, 1, BLK)
    y2 = y.astype(jnp.int32).reshape(1, B)
    pooled, loss11 = _pool_loss(parts, hists, batch3, y2, b.reshape(1, C), B, BLK)
    return pooled, loss11[0, 0]
